# baseline (speedup 1.0000x reference)
import functools

import jax
import jax.numpy as jnp
from jax import lax
from jax.experimental import pallas as pl
from jax.experimental.pallas import tpu as pltpu
from jax.experimental.pallas import tpu_sc as plsc

BATCH = 16384
N_CAT = 26
N_CONT = 26
VOCAB = 100000
EMB_DIM = 16
OUT_D = N_CAT * EMB_DIM + N_CONT

NW = 32
N_ROWS = N_CAT * EMB_DIM
ROWS_PER_W = N_ROWS // NW
HALF = 49920
HIGH = VOCAB - HALF
ICH = 8192
N_ICH = BATCH // ICH

_mesh = plsc.VectorSubcoreMesh(core_axis_name="c", subcore_axis_name="s")


@functools.partial(
    pl.kernel,
    mesh=_mesh,
    out_type=jax.ShapeDtypeStruct((OUT_D, BATCH), jnp.float32),
    scratch_types=[
        pltpu.VMEM((HIGH,), jnp.float32),
        pltpu.VMEM((HIGH,), jnp.float32),
        pltpu.VMEM((ICH,), jnp.int32),
        pltpu.VMEM((BATCH,), jnp.float32),
        pltpu.SemaphoreType.DMA,
        pltpu.SemaphoreType.DMA,
    ],
    compiler_params=pltpu.CompilerParams(needs_layout_passes=False),
)
def _emb_kernel(tab_hbm, idx_hbm, xtc_hbm, out_hbm, h0, h1, idx_v, out_v,
                sem0, sem1):
    w = lax.axis_index("s") * 2 + lax.axis_index("c")
    bufs = (h0, h1)
    sems = (sem0, sem1)

    def start(h):
        r0 = w * ROWS_PER_W + h // 2
        f0 = r0 // EMB_DIM
        e0 = r0 % EMB_DIM
        ln = HALF if h % 2 == 0 else HIGH
        return pltpu.async_copy(
            tab_hbm.at[f0, e0].at[pl.ds((h % 2) * HALF, ln)],
            bufs[h % 2].at[pl.ds(0, ln)], sems[h % 2])

    pending = start(0)
    for h in range(2 * ROWS_PER_W):
        nxt = start(h + 1) if h + 1 < 2 * ROWS_PER_W else None
        pending.wait()
        buf = bufs[h % 2]
        r = w * ROWS_PER_W + h // 2
        f = r // EMB_DIM
        lo_pass = (h % 2) == 0

        def _chunk(j, _):
            pltpu.sync_copy(idx_hbm.at[pl.ds(f * BATCH + j * ICH, ICH)], idx_v)

            def _groups(i, _):
                b = i * 128
                for u in range(8):
                    s = pl.ds(b + u * 16, 16)
                    o = pl.ds(j * ICH + b + u * 16, 16)
                    g = idx_v[s]
                    if lo_pass:
                        m = g < HALF
                        out_v[o] = plsc.load_gather(buf, [g], mask=m)
                    else:
                        m = g >= HALF
                        gh = plsc.load_gather(buf, [g - HALF], mask=m)
                        out_v[o] = jnp.where(m, gh, out_v[o])
                return 0

            lax.fori_loop(0, ICH // 128, _groups, 0)
            return 0

        lax.fori_loop(0, N_ICH, _chunk, 0)
        if not lo_pass:
            pltpu.sync_copy(out_v, out_hbm.at[r])
        pending = nxt

    @pl.when(w < N_CONT)
    def _cont():
        def _cchunk(j, _):
            pltpu.sync_copy(xtc_hbm.at[w, pl.ds(j * ICH, ICH)], idx_v)

            def _cgroups(i, _):
                b = i * 128
                for u in range(8):
                    s = pl.ds(b + u * 16, 16)
                    o = pl.ds(j * ICH + b + u * 16, 16)
                    out_v[o] = idx_v[s].astype(jnp.float32)
                return 0

            lax.fori_loop(0, ICH // 128, _cgroups, 0)
            return 0

        lax.fori_loop(0, N_ICH, _cchunk, 0)
        pltpu.sync_copy(out_v, out_hbm.at[N_ROWS + w])


def kernel(x, tables):
    tab_t = jnp.transpose(tables, (0, 2, 1))
    idx1d = jnp.transpose(x[:, :N_CAT]).reshape(-1)
    xtc = jnp.transpose(x[:, N_CAT:])
    out_t = _emb_kernel(tab_t, idx1d, xtc)
    return jnp.transpose(out_t)


# speedup vs baseline: 1.0108x; 1.0108x over previous
import functools

import jax
import jax.numpy as jnp
from jax import lax
from jax.experimental import pallas as pl
from jax.experimental.pallas import tpu as pltpu
from jax.experimental.pallas import tpu_sc as plsc

BATCH = 16384
N_CAT = 26
N_CONT = 26
VOCAB = 100000
EMB_DIM = 16
OUT_D = N_CAT * EMB_DIM + N_CONT

NW = 32
N_ROWS = N_CAT * EMB_DIM
ROWS_PER_W = N_ROWS // NW
HALF = 49920
HIGH = VOCAB - HALF
ICH = 8192
N_ICH = BATCH // ICH

_mesh = plsc.VectorSubcoreMesh(core_axis_name="c", subcore_axis_name="s")


@functools.partial(
    pl.kernel,
    mesh=_mesh,
    out_type=jax.ShapeDtypeStruct((OUT_D, BATCH), jnp.float32),
    scratch_types=[
        pltpu.VMEM((HIGH,), jnp.float32),
        pltpu.VMEM((HIGH,), jnp.float32),
        pltpu.VMEM((ICH,), jnp.int32),
        pltpu.VMEM((BATCH,), jnp.float32),
        pltpu.SemaphoreType.DMA,
        pltpu.SemaphoreType.DMA,
    ],
    compiler_params=pltpu.CompilerParams(needs_layout_passes=False),
)
def _emb_kernel(tab_hbm, idx_hbm, xtc_hbm, out_hbm, h0, h1, idx_v, out_v,
                sem0, sem1):
    w = lax.axis_index("s") * 2 + lax.axis_index("c")
    bufs = (h0, h1)
    sems = (sem0, sem1)

    def start(h):
        r0 = w * ROWS_PER_W + h // 2
        f0 = r0 // EMB_DIM
        e0 = r0 % EMB_DIM
        ln = HALF if h % 2 == 0 else HIGH
        return pltpu.async_copy(
            tab_hbm.at[f0, e0].at[pl.ds((h % 2) * HALF, ln)],
            bufs[h % 2].at[pl.ds(0, ln)], sems[h % 2])

    pending = start(0)
    for h in range(2 * ROWS_PER_W):
        nxt = start(h + 1) if h + 1 < 2 * ROWS_PER_W else None
        pending.wait()
        buf = bufs[h % 2]
        r = w * ROWS_PER_W + h // 2
        f = r // EMB_DIM
        lo_pass = (h % 2) == 0

        def _chunk(j, _):
            pltpu.sync_copy(idx_hbm.at[pl.ds(f * BATCH + j * ICH, ICH)], idx_v)

            lanes = lax.iota(jnp.int32, 16)

            def _groups(i, _):
                b = i * 128
                for u in range(8):
                    s = pl.ds(b + u * 16, 16)
                    g = idx_v[s]
                    if lo_pass:
                        m = g < HALF
                        out_v[pl.ds(j * ICH + b + u * 16, 16)] = (
                            plsc.load_gather(buf, [g], mask=m))
                    else:
                        m = g >= HALF
                        gh = plsc.load_gather(buf, [g - HALF], mask=m)
                        pos = lanes + (j * ICH + b + u * 16)
                        plsc.store_scatter(out_v, [pos], gh, mask=m)
                return 0

            lax.fori_loop(0, ICH // 128, _groups, 0)
            return 0

        lax.fori_loop(0, N_ICH, _chunk, 0)
        if not lo_pass:
            pltpu.sync_copy(out_v, out_hbm.at[r])
        pending = nxt

    @pl.when(w < N_CONT)
    def _cont():
        def _cchunk(j, _):
            pltpu.sync_copy(xtc_hbm.at[w, pl.ds(j * ICH, ICH)], idx_v)

            def _cgroups(i, _):
                b = i * 128
                for u in range(8):
                    s = pl.ds(b + u * 16, 16)
                    o = pl.ds(j * ICH + b + u * 16, 16)
                    out_v[o] = idx_v[s].astype(jnp.float32)
                return 0

            lax.fori_loop(0, ICH // 128, _cgroups, 0)
            return 0

        lax.fori_loop(0, N_ICH, _cchunk, 0)
        pltpu.sync_copy(out_v, out_hbm.at[N_ROWS + w])


def kernel(x, tables):
    tab_t = jnp.transpose(tables, (0, 2, 1))
    idx1d = jnp.transpose(x[:, :N_CAT]).reshape(-1)
    xtc = jnp.transpose(x[:, N_CAT:])
    out_t = _emb_kernel(tab_t, idx1d, xtc)
    return jnp.transpose(out_t)


# speedup vs baseline: 2.0245x; 2.0028x over previous
import functools

import jax
import jax.numpy as jnp
from jax import lax
from jax.experimental import pallas as pl
from jax.experimental.pallas import tpu as pltpu
from jax.experimental.pallas import tpu_sc as plsc

BATCH = 16384
N_CAT = 26
N_CONT = 26
VOCAB = 100000
EMB_DIM = 16
OUT_D = N_CAT * EMB_DIM + N_CONT

NW = 32
N_ROWS = N_CAT * EMB_DIM
ROWS_PER_W = N_ROWS // NW
HALF = 49920
HIGH = VOCAB - HALF
ICH = 8192
N_ICH = BATCH // ICH

_mesh = plsc.VectorSubcoreMesh(core_axis_name="c", subcore_axis_name="s")


@functools.partial(
    pl.kernel,
    mesh=_mesh,
    out_type=jax.ShapeDtypeStruct((OUT_D, BATCH), jnp.float32),
    scratch_types=[
        pltpu.VMEM((HIGH,), jnp.float32),
        pltpu.VMEM((HIGH,), jnp.float32),
        pltpu.VMEM((ICH,), jnp.int32),
        pltpu.VMEM((BATCH,), jnp.float32),
        pltpu.SemaphoreType.DMA,
        pltpu.SemaphoreType.DMA,
    ],
    compiler_params=pltpu.CompilerParams(needs_layout_passes=False),
)
def _emb_kernel(tab_hbm, idx_hbm, xtc_hbm, out_hbm, h0, h1, idx_v, out_v,
                sem0, sem1):
    w = lax.axis_index("s") * 2 + lax.axis_index("c")
    bufs = (h0, h1)
    sems = (sem0, sem1)

    def start(h):
        r0 = w * ROWS_PER_W + h // 2
        f0 = r0 // EMB_DIM
        e0 = r0 % EMB_DIM
        ln = HALF if h % 2 == 0 else HIGH
        return pltpu.async_copy(
            tab_hbm.at[f0, e0].at[pl.ds((h % 2) * HALF, ln)],
            bufs[h % 2].at[pl.ds(0, ln)], sems[h % 2])

    pending = start(0)
    for h in range(2 * ROWS_PER_W):
        nxt = start(h + 1) if h + 1 < 2 * ROWS_PER_W else None
        pending.wait()
        buf = bufs[h % 2]
        r = w * ROWS_PER_W + h // 2
        f = r // EMB_DIM
        lo_pass = (h % 2) == 0

        def _chunk(j, _):
            pltpu.sync_copy(idx_hbm.at[pl.ds(f * BATCH + j * ICH, ICH)], idx_v)

            lanes = lax.iota(jnp.int32, 16)

            if lo_pass:

                @plsc.parallel_loop(0, ICH // 16, unroll=8)
                def _groups(i):
                    g = idx_v[pl.ds(i * 16, 16)]
                    m = g < HALF
                    out_v[pl.ds(j * ICH + i * 16, 16)] = (
                        plsc.load_gather(buf, [g], mask=m))

            else:

                @plsc.parallel_loop(0, ICH // 16, unroll=8)
                def _groups(i):
                    g = idx_v[pl.ds(i * 16, 16)]
                    m = g >= HALF
                    gh = plsc.load_gather(buf, [g - HALF], mask=m)
                    plsc.store_scatter(
                        out_v, [lanes + (j * ICH + i * 16)], gh, mask=m)

            return 0

        lax.fori_loop(0, N_ICH, _chunk, 0)
        if not lo_pass:
            pltpu.sync_copy(out_v, out_hbm.at[r])
        pending = nxt

    @pl.when(w < N_CONT)
    def _cont():
        def _cchunk(j, _):
            pltpu.sync_copy(xtc_hbm.at[w, pl.ds(j * ICH, ICH)], idx_v)

            @plsc.parallel_loop(0, ICH // 16, unroll=8)
            def _cgroups(i):
                out_v[pl.ds(j * ICH + i * 16, 16)] = (
                    idx_v[pl.ds(i * 16, 16)].astype(jnp.float32))

            return 0

        lax.fori_loop(0, N_ICH, _cchunk, 0)
        pltpu.sync_copy(out_v, out_hbm.at[N_ROWS + w])


def kernel(x, tables):
    tab_t = jnp.transpose(tables, (0, 2, 1))
    idx1d = jnp.transpose(x[:, :N_CAT]).reshape(-1)
    xtc = jnp.transpose(x[:, N_CAT:])
    out_t = _emb_kernel(tab_t, idx1d, xtc)
    return jnp.transpose(out_t)


# speedup vs baseline: 2.0425x; 1.0089x over previous
import functools

import jax
import jax.numpy as jnp
from jax import lax
from jax.experimental import pallas as pl
from jax.experimental.pallas import tpu as pltpu
from jax.experimental.pallas import tpu_sc as plsc

BATCH = 16384
N_CAT = 26
N_CONT = 26
VOCAB = 100000
EMB_DIM = 16
OUT_D = N_CAT * EMB_DIM + N_CONT

NW = 32
N_ROWS = N_CAT * EMB_DIM
ROWS_PER_W = N_ROWS // NW
HALF = 49920
HIGH = VOCAB - HALF
ICH = 8192
N_ICH = BATCH // ICH

_mesh = plsc.VectorSubcoreMesh(core_axis_name="c", subcore_axis_name="s")


@functools.partial(
    pl.kernel,
    mesh=_mesh,
    out_type=jax.ShapeDtypeStruct((OUT_D, BATCH), jnp.float32),
    scratch_types=[
        pltpu.VMEM((HIGH,), jnp.float32),
        pltpu.VMEM((HIGH,), jnp.float32),
        pltpu.VMEM((ICH,), jnp.int32),
        pltpu.VMEM((BATCH,), jnp.float32),
        pltpu.SemaphoreType.DMA,
        pltpu.SemaphoreType.DMA,
    ],
    compiler_params=pltpu.CompilerParams(needs_layout_passes=False),
)
def _emb_kernel(tab_hbm, xt_hbm, out_hbm, h0, h1, idx_v, out_v,
                sem0, sem1):
    w = lax.axis_index("s") * 2 + lax.axis_index("c")
    bufs = (h0, h1)
    sems = (sem0, sem1)

    def start(h):
        r0 = w * ROWS_PER_W + h // 2
        f0 = r0 // EMB_DIM
        e0 = r0 % EMB_DIM
        ln = HALF if h % 2 == 0 else HIGH
        return pltpu.async_copy(
            tab_hbm.at[f0, e0].at[pl.ds((h % 2) * HALF, ln)],
            bufs[h % 2].at[pl.ds(0, ln)], sems[h % 2])

    pending = start(0)
    for h in range(2 * ROWS_PER_W):
        nxt = start(h + 1) if h + 1 < 2 * ROWS_PER_W else None
        pending.wait()
        buf = bufs[h % 2]
        r = w * ROWS_PER_W + h // 2
        f = r // EMB_DIM
        lo_pass = (h % 2) == 0

        def _chunk(j, _):
            pltpu.sync_copy(xt_hbm.at[f].at[pl.ds(j * ICH, ICH)], idx_v)

            lanes = lax.iota(jnp.int32, 16)

            if lo_pass:

                @plsc.parallel_loop(0, ICH // 16, unroll=8)
                def _groups(i):
                    g = idx_v[pl.ds(i * 16, 16)]
                    m = g < HALF
                    out_v[pl.ds(j * ICH + i * 16, 16)] = (
                        plsc.load_gather(buf, [g], mask=m))

            else:

                @plsc.parallel_loop(0, ICH // 16, unroll=8)
                def _groups(i):
                    g = idx_v[pl.ds(i * 16, 16)]
                    m = g >= HALF
                    gh = plsc.load_gather(buf, [g - HALF], mask=m)
                    plsc.store_scatter(
                        out_v, [lanes + (j * ICH + i * 16)], gh, mask=m)

            return 0

        lax.fori_loop(0, N_ICH, _chunk, 0)
        if not lo_pass:
            pltpu.sync_copy(out_v, out_hbm.at[r])
        pending = nxt

    @pl.when(w < N_CONT)
    def _cont():
        def _cchunk(j, _):
            pltpu.sync_copy(xt_hbm.at[N_CAT + w].at[pl.ds(j * ICH, ICH)], idx_v)

            @plsc.parallel_loop(0, ICH // 16, unroll=8)
            def _cgroups(i):
                out_v[pl.ds(j * ICH + i * 16, 16)] = (
                    idx_v[pl.ds(i * 16, 16)].astype(jnp.float32))

            return 0

        lax.fori_loop(0, N_ICH, _cchunk, 0)
        pltpu.sync_copy(out_v, out_hbm.at[N_ROWS + w])


def kernel(x, tables):
    tab_t = jnp.transpose(tables, (0, 2, 1))
    xt = jnp.transpose(x)
    out_t = _emb_kernel(tab_t, xt)
    return jnp.transpose(out_t)


# speedup vs baseline: 2.4298x; 1.1896x over previous
import functools

import jax
import jax.numpy as jnp
from jax import lax
from jax.experimental import pallas as pl
from jax.experimental.pallas import tpu as pltpu
from jax.experimental.pallas import tpu_sc as plsc

BATCH = 16384
N_CAT = 26
N_CONT = 26
VOCAB = 100000
EMB_DIM = 16
OUT_D = N_CAT * EMB_DIM + N_CONT

NW = 32
N_ROWS = N_CAT * EMB_DIM
ROWS_PER_W = N_ROWS // NW
HALF = 49920
HIGH = VOCAB - HALF
ICH = 4096
N_ICH = BATCH // ICH

_mesh = plsc.VectorSubcoreMesh(core_axis_name="c", subcore_axis_name="s")


@functools.partial(
    pl.kernel,
    mesh=_mesh,
    out_type=jax.ShapeDtypeStruct((OUT_D, BATCH), jnp.float32),
    scratch_types=[
        pltpu.VMEM((HALF,), jnp.float32),
        pltpu.VMEM((HIGH,), jnp.float32),
        pltpu.VMEM((ICH,), jnp.int32),
        pltpu.VMEM((ICH,), jnp.int32),
        pltpu.VMEM((BATCH,), jnp.float32),
        pltpu.SemaphoreType.DMA,
        pltpu.SemaphoreType.DMA,
        pltpu.SemaphoreType.DMA,
        pltpu.SemaphoreType.DMA,
    ],
    compiler_params=pltpu.CompilerParams(needs_layout_passes=False),
)
def _emb_kernel(tab_hbm, xt_hbm, out_hbm, blo, bhi, ia, ib, out_v,
                slo, shi, sa, sb):
    w = lax.axis_index("s") * 2 + lax.axis_index("c")
    ibufs = (ia, ib)
    isems = (sa, sb)
    lanes = lax.iota(jnp.int32, 16)

    def fe(rr):
        return rr // EMB_DIM, rr % EMB_DIM

    r0 = w * ROWS_PER_W
    f0, e0 = fe(r0)
    pltpu.async_copy(tab_hbm.at[f0, e0].at[pl.ds(0, HALF)], blo, slo)
    pltpu.async_copy(xt_hbm.at[f0].at[pl.ds(0, ICH)], ia, sa)

    def _task(k, _):
        r = w * ROWS_PER_W + k
        f, e = fe(r)
        row = tab_hbm.at[f, e]
        pltpu.async_copy(row.at[pl.ds(HALF, HIGH)], bhi, shi)
        pltpu.make_async_copy(row.at[pl.ds(0, HALF)], blo, slo).wait()

        for c8 in range(2 * N_ICH):
            jc = c8 % N_ICH
            par = c8 % 2
            if c8 < 2 * N_ICH - 1:
                njc = (c8 + 1) % N_ICH
                pltpu.async_copy(
                    xt_hbm.at[f].at[pl.ds(njc * ICH, ICH)],
                    ibufs[1 - par], isems[1 - par])
            else:

                @pl.when(k < ROWS_PER_W - 1)
                def _():
                    fn, _en = fe(r + 1)
                    pltpu.async_copy(
                        xt_hbm.at[fn].at[pl.ds(0, ICH)], ibufs[0], isems[0])

            if c8 == N_ICH:
                @pl.when(k < ROWS_PER_W - 1)
                def _():
                    fn, en = fe(r + 1)
                    pltpu.async_copy(
                        tab_hbm.at[fn, en].at[pl.ds(0, HALF)], blo, slo)

                pltpu.make_async_copy(row.at[pl.ds(HALF, HIGH)],
                                      bhi, shi).wait()

            idx_v = ibufs[par]
            pltpu.make_async_copy(xt_hbm.at[f].at[pl.ds(jc * ICH, ICH)],
                                  idx_v, isems[par]).wait()

            if c8 < N_ICH:

                @plsc.parallel_loop(0, ICH // 16, unroll=8)
                def _groups(i):
                    g = idx_v[pl.ds(i * 16, 16)]
                    m = g < HALF
                    out_v[pl.ds(jc * ICH + i * 16, 16)] = (
                        plsc.load_gather(blo, [g], mask=m))

            else:

                @plsc.parallel_loop(0, ICH // 16, unroll=8)
                def _groups(i):
                    g = idx_v[pl.ds(i * 16, 16)]
                    m = g >= HALF
                    gh = plsc.load_gather(bhi, [g - HALF], mask=m)
                    plsc.store_scatter(
                        out_v, [lanes + (jc * ICH + i * 16)], gh, mask=m)

        pltpu.sync_copy(out_v, out_hbm.at[r])
        return 0

    lax.fori_loop(0, ROWS_PER_W, _task, 0)

    @pl.when(w < N_CONT)
    def _cont():
        def _cchunk(j, _):
            pltpu.sync_copy(xt_hbm.at[N_CAT + w].at[pl.ds(j * ICH, ICH)], ia)

            @plsc.parallel_loop(0, ICH // 16, unroll=8)
            def _cgroups(i):
                out_v[pl.ds(j * ICH + i * 16, 16)] = (
                    ia[pl.ds(i * 16, 16)].astype(jnp.float32))

            return 0

        lax.fori_loop(0, N_ICH, _cchunk, 0)
        pltpu.sync_copy(out_v, out_hbm.at[N_ROWS + w])


def kernel(x, tables):
    tab_t = jnp.transpose(tables, (0, 2, 1))
    xt = jnp.transpose(x)
    out_t = _emb_kernel(tab_t, xt)
    return jnp.transpose(out_t)
